# baseline (speedup 1.0000x reference)
import functools

import jax
import jax.numpy as jnp
from jax import lax
from jax.experimental import pallas as pl
from jax.experimental.pallas import tpu as pltpu
from jax.experimental.pallas import tpu_sc as plsc

N = 10000
E = 320000
NC = 2
NS = 16
NW = NC * NS
CH = 40
EPT = E // NW
NCHUNK = EPT // CH
EPAD = NW * EPT
NPAD = 10240
RPT = NPAD // NS
DUMMY_DST = N

_MESH = plsc.VectorSubcoreMesh(core_axis_name="c", subcore_axis_name="s")


def _make_msg_kernel(d):
    @functools.partial(
        pl.kernel,
        mesh=_MESH,
        out_type=jax.ShapeDtypeStruct((NC, NPAD, d), jnp.float32),
        scratch_types=[
            pltpu.VMEM((CH,), jnp.int32),
            pltpu.VMEM((CH,), jnp.int32),
            pltpu.VMEM((CH, d), jnp.float32),
            pltpu.VMEM_SHARED((NPAD, d), jnp.float32),
            pltpu.SemaphoreType.DMA,
        ],
    )
    def msg_kernel(g_hbm, src_hbm, dst_hbm, zeros_hbm, out_hbm,
                   sb, db, rows, acc, sem):
        c = lax.axis_index("c")
        s = lax.axis_index("s")
        wid = c * NS + s
        base = wid * EPT

        pltpu.sync_copy(zeros_hbm.at[pl.ds(s * RPT, RPT)],
                        acc.at[pl.ds(s * RPT, RPT)])
        plsc.subcore_barrier()

        @pl.loop(0, NCHUNK)
        def _(i):
            off = base + i * CH
            pltpu.sync_copy(src_hbm.at[pl.ds(off, CH)], sb)
            pltpu.sync_copy(dst_hbm.at[pl.ds(off, CH)], db)
            pltpu.async_copy(g_hbm.at[sb], rows, sem).wait()
            pltpu.sync_copy(rows, acc.at[db], add=True)

        plsc.subcore_barrier()
        pltpu.sync_copy(acc.at[pl.ds(s * RPT, RPT)],
                        out_hbm.at[c, pl.ds(s * RPT, RPT)])

    return msg_kernel


_msg_kernel_128 = _make_msg_kernel(128)


def _tc0_body(deg_ref, x_ref, w_ref, dinv_ref, g_ref):
    deg = deg_ref[0, :N, 0:1] + deg_ref[1, :N, 0:1] + 1.0
    dinv = lax.rsqrt(deg)
    dinv_ref[...] = dinv
    h = jnp.dot(x_ref[...], w_ref[...], preferred_element_type=jnp.float32)
    g_ref[...] = h * dinv


def _tc_boundary_body(acc_ref, g_ref, dinv_ref, b_ref, w_ref, gn_ref):
    dinv = dinv_ref[...]
    m = acc_ref[0, :N, :] + acc_ref[1, :N, :] + g_ref[...]
    a = jnp.maximum(m * dinv + b_ref[...], 0.0)
    gn_ref[...] = jnp.dot(a, w_ref[...],
                          preferred_element_type=jnp.float32) * dinv


def _tc_final_body(acc_ref, g_ref, dinv_ref, b_ref, out_ref):
    m = acc_ref[0, :N, :] + acc_ref[1, :N, :] + g_ref[...]
    out_ref[...] = m * dinv_ref[...] + b_ref[...]


def _tc0(deg, x, w):
    return pl.pallas_call(
        _tc0_body,
        out_shape=(jax.ShapeDtypeStruct((N, 1), jnp.float32),
                   jax.ShapeDtypeStruct((N, w.shape[1]), jnp.float32)),
    )(deg, x, w)


def _tc_boundary(acc, g, dinv, b, w):
    return pl.pallas_call(
        _tc_boundary_body,
        out_shape=jax.ShapeDtypeStruct((N, w.shape[1]), jnp.float32),
    )(acc, g, dinv, b.reshape(1, -1), w)


def _tc_final(acc, g, dinv, b):
    return pl.pallas_call(
        _tc_final_body,
        out_shape=jax.ShapeDtypeStruct((N, g.shape[1]), jnp.float32),
    )(acc, g, dinv, b.reshape(1, -1))


def kernel(x, edge_index, W1, b1, W2, b2, W3, b3, W4, b4):
    ei = edge_index.astype(jnp.int32)
    npad_e = EPAD - E
    src = jnp.concatenate([ei[0], jnp.zeros((npad_e,), jnp.int32)])
    pad_dst = DUMMY_DST + (jnp.arange(npad_e, dtype=jnp.int32) % (NPAD - N))
    dst = jnp.concatenate([ei[1], pad_dst])
    zeros128 = jnp.zeros((NPAD, 128), jnp.float32)
    ones128 = jnp.ones((N, 128), jnp.float32)

    W2p = jnp.pad(W2, ((0, 0), (0, 64)))
    b2p = jnp.pad(b2, (0, 64))
    W3p = jnp.pad(W3, ((0, 64), (0, 0)))

    deg = _msg_kernel_128(ones128, src, dst, zeros128)
    dinv, g1 = _tc0(deg, x, W1)

    acc1 = _msg_kernel_128(g1, src, dst, zeros128)
    g2 = _tc_boundary(acc1, g1, dinv, b1, W2p)

    acc2 = _msg_kernel_128(g2, src, dst, zeros128)
    g3 = _tc_boundary(acc2, g2, dinv, b2p, W3p)

    acc3 = _msg_kernel_128(g3, src, dst, zeros128)
    g4 = _tc_boundary(acc3, g3, dinv, b3, W4)

    acc4 = _msg_kernel_128(g4, src, dst, zeros128)
    return _tc_final(acc4, g4, dinv, b4)


# speedup vs baseline: 3.1732x; 3.1732x over previous
import functools

import jax
import jax.numpy as jnp
from jax import lax
from jax.experimental import pallas as pl
from jax.experimental.pallas import tpu as pltpu
from jax.experimental.pallas import tpu_sc as plsc

N = 10000
E = 320000
NC = 2
NS = 16
NW = NC * NS
CH = 80
EPT = E // NW
NCHUNK = EPT // CH
EPAD = NW * EPT
NPAD = 10240
RPT = NPAD // NS
DUMMY_DST = N

_MESH = plsc.VectorSubcoreMesh(core_axis_name="c", subcore_axis_name="s")


def _make_msg_kernel(d):
    @functools.partial(
        pl.kernel,
        mesh=_MESH,
        out_type=jax.ShapeDtypeStruct((NC, NPAD, d), jnp.float32),
        scratch_types=[
            pltpu.VMEM((CH,), jnp.int32),
            pltpu.VMEM((CH,), jnp.int32),
            pltpu.VMEM((CH,), jnp.int32),
            pltpu.VMEM((CH,), jnp.int32),
            pltpu.VMEM((CH, d), jnp.float32),
            pltpu.VMEM((CH, d), jnp.float32),
            pltpu.VMEM_SHARED((NPAD, d), jnp.float32),
            pltpu.SemaphoreType.DMA,
            pltpu.SemaphoreType.DMA,
            pltpu.SemaphoreType.DMA,
            pltpu.SemaphoreType.DMA,
            pltpu.SemaphoreType.DMA,
            pltpu.SemaphoreType.DMA,
        ],
    )
    def msg_kernel(g_hbm, src_hbm, dst_hbm, zeros_hbm, out_hbm,
                   sb0, sb1, db0, db1, rows0, rows1, acc,
                   si0, si1, di0, di1, sg0, sg1):
        c = lax.axis_index("c")
        s = lax.axis_index("s")
        wid = c * NS + s
        base = wid * EPT

        def fS(i, b, sem):
            return pltpu.make_async_copy(
                src_hbm.at[pl.ds(base + i * CH, CH)], b, sem)

        def fD(i, b, sem):
            return pltpu.make_async_copy(
                dst_hbm.at[pl.ds(base + i * CH, CH)], b, sem)

        def gat(b, r, sem):
            return pltpu.make_async_copy(g_hbm.at[b], r, sem)

        def sca(r, b):
            pltpu.sync_copy(r, acc.at[b], add=True)

        pltpu.sync_copy(zeros_hbm.at[pl.ds(s * RPT, RPT)],
                        acc.at[pl.ds(s * RPT, RPT)])
        plsc.subcore_barrier()

        fS(0, sb0, si0).start()
        fD(0, db0, di0).start()
        fS(1, sb1, si1).start()
        fD(1, db1, di1).start()
        fS(0, sb0, si0).wait()
        gat(sb0, rows0, sg0).start()

        @pl.loop(0, NCHUNK - 1, step=2)
        def _(i):
            fS(i + 1, sb1, si1).wait()
            gat(sb1, rows1, sg1).start()
            gat(sb0, rows0, sg0).wait()
            fD(i, db0, di0).wait()
            sca(rows0, db0)
            fS(i + 2, sb0, si0).start()
            fD(i + 2, db0, di0).start()
            fS(i + 2, sb0, si0).wait()
            gat(sb0, rows0, sg0).start()
            gat(sb1, rows1, sg1).wait()
            fD(i + 1, db1, di1).wait()
            sca(rows1, db1)
            fS(i + 3, sb1, si1).start()
            fD(i + 3, db1, di1).start()

        gat(sb0, rows0, sg0).wait()
        fD(NCHUNK - 1, db0, di0).wait()
        sca(rows0, db0)
        fS(NCHUNK, sb1, si1).wait()
        fD(NCHUNK, db1, di1).wait()

        plsc.subcore_barrier()
        pltpu.sync_copy(acc.at[pl.ds(s * RPT, RPT)],
                        out_hbm.at[c, pl.ds(s * RPT, RPT)])

    return msg_kernel


_msg_kernel_128 = _make_msg_kernel(128)


def _tc0_body(deg_ref, x_ref, w_ref, dinv_ref, g_ref):
    deg = deg_ref[0, :N, 0:1] + deg_ref[1, :N, 0:1] + 1.0
    dinv = lax.rsqrt(deg)
    dinv_ref[...] = dinv
    h = jnp.dot(x_ref[...], w_ref[...], preferred_element_type=jnp.float32)
    g_ref[...] = h * dinv


def _tc_boundary_body(acc_ref, g_ref, dinv_ref, b_ref, w_ref, gn_ref):
    dinv = dinv_ref[...]
    m = acc_ref[0, :N, :] + acc_ref[1, :N, :] + g_ref[...]
    a = jnp.maximum(m * dinv + b_ref[...], 0.0)
    gn_ref[...] = jnp.dot(a, w_ref[...],
                          preferred_element_type=jnp.float32) * dinv


def _tc_final_body(acc_ref, g_ref, dinv_ref, b_ref, out_ref):
    m = acc_ref[0, :N, :] + acc_ref[1, :N, :] + g_ref[...]
    out_ref[...] = m * dinv_ref[...] + b_ref[...]


def _tc0(deg, x, w):
    return pl.pallas_call(
        _tc0_body,
        out_shape=(jax.ShapeDtypeStruct((N, 1), jnp.float32),
                   jax.ShapeDtypeStruct((N, w.shape[1]), jnp.float32)),
    )(deg, x, w)


def _tc_boundary(acc, g, dinv, b, w):
    return pl.pallas_call(
        _tc_boundary_body,
        out_shape=jax.ShapeDtypeStruct((N, w.shape[1]), jnp.float32),
    )(acc, g, dinv, b.reshape(1, -1), w)


def _tc_final(acc, g, dinv, b):
    return pl.pallas_call(
        _tc_final_body,
        out_shape=jax.ShapeDtypeStruct((N, g.shape[1]), jnp.float32),
    )(acc, g, dinv, b.reshape(1, -1))


def kernel(x, edge_index, W1, b1, W2, b2, W3, b3, W4, b4):
    ei = edge_index.astype(jnp.int32)
    src = jnp.concatenate([ei[0], jnp.zeros((CH,), jnp.int32)])
    dst = jnp.concatenate([ei[1], jnp.full((CH,), DUMMY_DST, jnp.int32)])
    zeros128 = jnp.zeros((NPAD, 128), jnp.float32)
    ones128 = jnp.ones((N, 128), jnp.float32)

    W2p = jnp.pad(W2, ((0, 0), (0, 64)))
    b2p = jnp.pad(b2, (0, 64))
    W3p = jnp.pad(W3, ((0, 64), (0, 0)))

    deg = _msg_kernel_128(ones128, src, dst, zeros128)
    dinv, g1 = _tc0(deg, x, W1)

    acc1 = _msg_kernel_128(g1, src, dst, zeros128)
    g2 = _tc_boundary(acc1, g1, dinv, b1, W2p)

    acc2 = _msg_kernel_128(g2, src, dst, zeros128)
    g3 = _tc_boundary(acc2, g2, dinv, b2p, W3p)

    acc3 = _msg_kernel_128(g3, src, dst, zeros128)
    g4 = _tc_boundary(acc3, g3, dinv, b3, W4)

    acc4 = _msg_kernel_128(g4, src, dst, zeros128)
    return _tc_final(acc4, g4, dinv, b4)
